# baseline (speedup 1.0000x reference)
import functools

import jax
import jax.numpy as jnp
from jax import lax
from jax.experimental import pallas as pl
from jax.experimental.pallas import tpu as pltpu
from jax.experimental.pallas import tpu_sc as plsc

_B = 8
_N = 4096
_SKP = 256
_K1 = 32
_K2 = 8
_SP = 64
_SCALE = 1.5
_NC = 2
_L = 16


def _fps_sc_build(n_points, n_samples, with_aux):
    mesh = plsc.VectorSubcoreMesh(core_axis_name="c", subcore_axis_name="s")
    out_type = [jax.ShapeDtypeStruct((3 * _B, n_samples), jnp.float32)]
    if with_aux:
        out_type.append(jax.ShapeDtypeStruct((_B, n_samples), jnp.float32))
    scratch = [
        pltpu.VMEM((n_points,), jnp.float32),
        pltpu.VMEM((n_points,), jnp.float32),
        pltpu.VMEM((n_points,), jnp.float32),
        pltpu.VMEM((n_points,), jnp.float32),
        pltpu.VMEM((n_samples,), jnp.float32),
        pltpu.VMEM((n_samples,), jnp.float32),
        pltpu.VMEM((n_samples,), jnp.float32),
    ]
    if with_aux:
        scratch.append(pltpu.VMEM((n_points,), jnp.float32))
        scratch.append(pltpu.VMEM((n_samples,), jnp.float32))

    def body(*refs):
        if with_aux:
            (planes, aux_h, out_h, aux_out_h,
             x_v, y_v, z_v, dist_v, cx_v, cy_v, cz_v, a_v, ao_v) = refs
        else:
            (planes, out_h,
             x_v, y_v, z_v, dist_v, cx_v, cy_v, cz_v) = refs
        wid = lax.axis_index("s") * _NC + lax.axis_index("c")

        @pl.when(wid < _B)
        def _():
            b = wid
            pltpu.sync_copy(planes.at[3 * b + 0], x_v)
            pltpu.sync_copy(planes.at[3 * b + 1], y_v)
            pltpu.sync_copy(planes.at[3 * b + 2], z_v)
            if with_aux:
                pltpu.sync_copy(aux_h.at[b], a_v)
            lane = lax.iota(jnp.int32, _L)
            mask0 = lane == 0
            nchunks = n_points // _L

            def init_body(j, carry):
                dist_v[pl.ds(j * _L, _L)] = jnp.full((_L,), 1e10, jnp.float32)
                return carry

            lax.fori_loop(0, nchunks, init_body, 0)

            def put(i, src):
                iv = jnp.full((_L,), i, jnp.int32)
                sv = jnp.full((_L,), src, jnp.int32)
                plsc.store_scatter(cx_v, [iv], plsc.load_gather(x_v, [sv]),
                                   mask=mask0)
                plsc.store_scatter(cy_v, [iv], plsc.load_gather(y_v, [sv]),
                                   mask=mask0)
                plsc.store_scatter(cz_v, [iv], plsc.load_gather(z_v, [sv]),
                                   mask=mask0)
                if with_aux:
                    plsc.store_scatter(ao_v, [iv], plsc.load_gather(a_v, [sv]),
                                       mask=mask0)

            put(jnp.int32(0), jnp.int32(0))

            def step(i, last):
                lv = jnp.full((_L,), last, jnp.int32)
                xl = plsc.load_gather(x_v, [lv])
                yl = plsc.load_gather(y_v, [lv])
                zl = plsc.load_gather(z_v, [lv])

                def chunk(j, carry):
                    bval, bidx = carry
                    off = j * _L
                    dx = x_v[pl.ds(off, _L)] - xl
                    dy = y_v[pl.ds(off, _L)] - yl
                    dz = z_v[pl.ds(off, _L)] - zl
                    d = dx * dx + dy * dy
                    d = d + dz * dz
                    dm = jnp.minimum(dist_v[pl.ds(off, _L)], d)
                    dist_v[pl.ds(off, _L)] = dm
                    upd = dm > bval
                    return (jnp.where(upd, dm, bval),
                            jnp.where(upd, lane + off, bidx))

                bval, bidx = lax.fori_loop(
                    0, nchunks, chunk,
                    (jnp.full((_L,), -1.0, jnp.float32),
                     jnp.zeros((_L,), jnp.int32)))
                gmax = jnp.max(bval)
                nxt = jnp.min(jnp.where(bval == gmax, bidx,
                                        jnp.int32(n_points)))
                put(i, nxt)
                return nxt

            lax.fori_loop(1, n_samples, step, jnp.int32(0))
            pltpu.sync_copy(cx_v, out_h.at[3 * b + 0])
            pltpu.sync_copy(cy_v, out_h.at[3 * b + 1])
            pltpu.sync_copy(cz_v, out_h.at[3 * b + 2])
            if with_aux:
                pltpu.sync_copy(ao_v, aux_out_h.at[b])

    return functools.partial(
        pl.kernel, mesh=mesh, out_type=tuple(out_type) if with_aux
        else out_type[0], scratch_types=scratch,
        compiler_params=pltpu.CompilerParams(needs_layout_passes=False))(body)


def _fps_sc(xyz, n_samples, aux=None):
    n_points = xyz.shape[1]
    planes = jnp.transpose(xyz, (0, 2, 1)).reshape(3 * _B, n_points)
    if aux is None:
        out = _fps_sc_build(n_points, n_samples, False)(planes)
        outs = (out,)
    else:
        outs = _fps_sc_build(n_points, n_samples, True)(planes, aux)
    samp = outs[0].reshape(_B, 3, n_samples).transpose(0, 2, 1)
    if aux is None:
        return samp
    return samp, outs[1]


def _skel_compute(pts_ref, pT_ref, c_ref, d2_ref, nbr_ref, *, k, n):
    px = pT_ref[0, 0:1, :]
    py = pT_ref[0, 1:2, :]
    pz = pT_ref[0, 2:3, :]
    cen = c_ref[0]
    cx = cen[:, 0:1]
    cy = cen[:, 1:2]
    cz = cen[:, 2:3]
    dx = cx - px
    dy = cy - py
    dz = cz - pz
    d2 = dx * dx + dy * dy
    d2 = d2 + dz * dz
    mb = d2.shape[0]
    d2_ref[...] = d2
    nh = 4
    mh = mb // nh
    iota_h = lax.broadcasted_iota(jnp.int32, (mh, n), 1)

    def select_pass(j, nsums):
        nbrs = []
        for h in range(nh):
            rows = pl.ds(h * mh, mh)
            d2c = d2_ref[rows, :]
            mval = jnp.min(d2c, axis=1, keepdims=True)
            idx = jnp.min(jnp.where(d2c == mval, iota_h, jnp.int32(n)),
                          axis=1, keepdims=True)
            oh = iota_h == idx
            d2_ref[rows, :] = jnp.where(oh, jnp.float32(jnp.inf), d2c)
            ohf = oh.astype(jnp.float32)
            nbr = lax.dot_general(ohf, pts_ref[0], (((1,), (0,)), ((), ())),
                                  precision=lax.Precision.HIGHEST)
            nbrs.append(nbr)
        nbr_ref[j] = jnp.concatenate(nbrs, axis=0)
        return tuple(ns + nb for ns, nb in zip(nsums, nbrs))

    nsums = lax.fori_loop(
        0, k, select_pass,
        tuple(jnp.zeros((mh, 3), jnp.float32) for _ in range(nh)))
    newc = jnp.concatenate(nsums, axis=0) / jnp.float32(k)

    def rad_pass(j, rsum):
        e = nbr_ref[j] - newc
        dd = e[:, 0:1] * e[:, 0:1] + e[:, 1:2] * e[:, 1:2]
        dd = dd + e[:, 2:3] * e[:, 2:3]
        return rsum + jnp.sqrt(dd + 1e-12)

    rad = lax.fori_loop(0, k, rad_pass,
                        jnp.zeros((mb, 1), jnp.float32)) / jnp.float32(k)
    return newc, rad


def _skel_body(pts_ref, pT_ref, c_ref, oc_ref, or_ref, d2_ref, nbr_ref,
               *, k, n):
    newc, rad = _skel_compute(pts_ref, pT_ref, c_ref, d2_ref, nbr_ref,
                              k=k, n=n)
    oc_ref[0] = newc
    or_ref[0] = rad


def _skel_tc(points, points_tp, centers, k, mb):
    b, n, _ = points.shape
    m = centers.shape[1]
    grid = (b, m // mb)
    return pl.pallas_call(
        functools.partial(_skel_body, k=k, n=n),
        grid=grid,
        in_specs=[
            pl.BlockSpec((1, n, 3), lambda i, j: (i, 0, 0)),
            pl.BlockSpec((1, 8, n), lambda i, j: (i, 0, 0)),
            pl.BlockSpec((1, mb, 3), lambda i, j: (i, j, 0)),
        ],
        out_specs=[
            pl.BlockSpec((1, mb, 3), lambda i, j: (i, j, 0)),
            pl.BlockSpec((1, mb, 1), lambda i, j: (i, j, 0)),
        ],
        out_shape=[
            jax.ShapeDtypeStruct((b, m, 3), jnp.float32),
            jax.ShapeDtypeStruct((b, m, 1), jnp.float32),
        ],
        scratch_shapes=[
            pltpu.VMEM((mb, n), jnp.float32),
            pltpu.VMEM((k, mb, 3), jnp.float32),
        ],
    )(points, points_tp, centers)


def _skel_final_body(pts_ref, pT_ref, c_ref, rg_ref, u_ref,
                     oc_ref, or_ref, os_ref, d2_ref, nbr_ref, *, k, n):
    newc, rad = _skel_compute(pts_ref, pT_ref, c_ref, d2_ref, nbr_ref,
                              k=k, n=n)
    oc_ref[0] = newc
    radf = (rg_ref[0] + rad) * jnp.float32(_SCALE)
    or_ref[0] = radf
    cexp = jnp.concatenate([newc] * _SP, axis=1)
    uexp = u_ref[0:1, :]
    os_ref[0] = cexp + radf * uexp


def _fibonacci_sphere(npts):
    i = jnp.arange(npts, dtype=jnp.float32)
    golden = jnp.pi * (3.0 - jnp.sqrt(5.0))
    y = 1.0 - 2.0 * (i + 0.5) / npts
    r = jnp.sqrt(jnp.clip(1.0 - y * y, 0.0, 1.0))
    theta = golden * i
    return jnp.stack([r * jnp.cos(theta), y, r * jnp.sin(theta)], axis=-1)


def _skel_final_tc(points, points_tp, centers, rgath, k, mb):
    b, n, _ = points.shape
    m = centers.shape[1]
    grid = (b, m // mb)
    uflat = jnp.tile(_fibonacci_sphere(_SP).reshape(1, 3 * _SP), (8, 1))
    return pl.pallas_call(
        functools.partial(_skel_final_body, k=k, n=n),
        grid=grid,
        in_specs=[
            pl.BlockSpec((1, n, 3), lambda i, j: (i, 0, 0)),
            pl.BlockSpec((1, 8, n), lambda i, j: (i, 0, 0)),
            pl.BlockSpec((1, mb, 3), lambda i, j: (i, j, 0)),
            pl.BlockSpec((1, mb, 1), lambda i, j: (i, j, 0)),
            pl.BlockSpec((8, 3 * _SP), lambda i, j: (0, 0)),
        ],
        out_specs=[
            pl.BlockSpec((1, mb, 3), lambda i, j: (i, j, 0)),
            pl.BlockSpec((1, mb, 1), lambda i, j: (i, j, 0)),
            pl.BlockSpec((1, mb, 3 * _SP), lambda i, j: (i, j, 0)),
        ],
        out_shape=[
            jax.ShapeDtypeStruct((b, m, 3), jnp.float32),
            jax.ShapeDtypeStruct((b, m, 1), jnp.float32),
            jax.ShapeDtypeStruct((b, m, 3 * _SP), jnp.float32),
        ],
        scratch_shapes=[
            pltpu.VMEM((mb, n), jnp.float32),
            pltpu.VMEM((k, mb, 3), jnp.float32),
        ],
    )(points, points_tp, centers, rgath, uflat)


def _pad_tp(points):
    tp = jnp.transpose(points, (0, 2, 1))
    return jnp.pad(tp, ((0, 0), (0, 5), (0, 0)))


def kernel(xyz):
    c1 = _fps_sc(xyz, _SKP * 4)
    xyz_tp = _pad_tp(xyz)
    nc1, _ = _skel_tc(xyz, xyz_tp, c1, _K1, 256)
    xyz2, rad1 = _skel_tc(xyz, xyz_tp, nc1, _K1, 256)
    c2, rg = _fps_sc(xyz2, _SKP, aux=rad1.reshape(_B, -1))
    xyz2_tp = _pad_tp(xyz2)
    nc2, _ = _skel_tc(xyz2, xyz2_tp, c2, _K2, 256)
    centers, radius, sflat = _skel_final_tc(
        xyz2, xyz2_tp, nc2, rg.reshape(_B, _SKP, 1), _K2, 256)
    sphere = sflat.reshape(_B, _SKP, _SP, 3)
    return centers, sphere, radius


# speedup vs baseline: 1.8706x; 1.8706x over previous
import functools

import jax
import jax.numpy as jnp
from jax import lax
from jax.experimental import pallas as pl
from jax.experimental.pallas import tpu as pltpu
from jax.experimental.pallas import tpu_sc as plsc

_B = 8
_N = 4096
_SKP = 256
_K1 = 32
_K2 = 8
_SP = 64
_SCALE = 1.5
_NC = 2
_L = 16


def _fps_sc_build(n_points, n_samples, with_aux):
    mesh = plsc.VectorSubcoreMesh(core_axis_name="c", subcore_axis_name="s")
    out_type = [jax.ShapeDtypeStruct((3 * _B, n_samples), jnp.float32)]
    if with_aux:
        out_type.append(jax.ShapeDtypeStruct((_B, n_samples), jnp.float32))
    scratch = [
        pltpu.VMEM((n_points,), jnp.float32),
        pltpu.VMEM((n_points,), jnp.float32),
        pltpu.VMEM((n_points,), jnp.float32),
        pltpu.VMEM((n_points,), jnp.float32),
        pltpu.VMEM((n_samples,), jnp.float32),
        pltpu.VMEM((n_samples,), jnp.float32),
        pltpu.VMEM((n_samples,), jnp.float32),
    ]
    if with_aux:
        scratch.append(pltpu.VMEM((n_points,), jnp.float32))
        scratch.append(pltpu.VMEM((n_samples,), jnp.float32))

    def body(*refs):
        if with_aux:
            (planes, aux_h, out_h, aux_out_h,
             x_v, y_v, z_v, dist_v, cx_v, cy_v, cz_v, a_v, ao_v) = refs
        else:
            (planes, out_h,
             x_v, y_v, z_v, dist_v, cx_v, cy_v, cz_v) = refs
        wid = lax.axis_index("s") * _NC + lax.axis_index("c")

        @pl.when(wid < _B)
        def _():
            b = wid
            pltpu.sync_copy(planes.at[3 * b + 0], x_v)
            pltpu.sync_copy(planes.at[3 * b + 1], y_v)
            pltpu.sync_copy(planes.at[3 * b + 2], z_v)
            if with_aux:
                pltpu.sync_copy(aux_h.at[b], a_v)
            lane = lax.iota(jnp.int32, _L)
            mask0 = lane == 0
            nchunks = n_points // _L

            def init_body(j, carry):
                dist_v[pl.ds(j * _L, _L)] = jnp.full((_L,), 1e10, jnp.float32)
                return carry

            lax.fori_loop(0, nchunks, init_body, 0)

            def put(i, src):
                iv = jnp.full((_L,), i, jnp.int32)
                sv = jnp.full((_L,), src, jnp.int32)
                plsc.store_scatter(cx_v, [iv], plsc.load_gather(x_v, [sv]),
                                   mask=mask0)
                plsc.store_scatter(cy_v, [iv], plsc.load_gather(y_v, [sv]),
                                   mask=mask0)
                plsc.store_scatter(cz_v, [iv], plsc.load_gather(z_v, [sv]),
                                   mask=mask0)
                if with_aux:
                    plsc.store_scatter(ao_v, [iv], plsc.load_gather(a_v, [sv]),
                                       mask=mask0)

            put(jnp.int32(0), jnp.int32(0))

            def step(i, last):
                lv = jnp.full((_L,), last, jnp.int32)
                xl = plsc.load_gather(x_v, [lv])
                yl = plsc.load_gather(y_v, [lv])
                zl = plsc.load_gather(z_v, [lv])

                def chunk(j, carry):
                    bval, bidx = carry
                    off = j * _L
                    dx = x_v[pl.ds(off, _L)] - xl
                    dy = y_v[pl.ds(off, _L)] - yl
                    dz = z_v[pl.ds(off, _L)] - zl
                    d = dx * dx + dy * dy
                    d = d + dz * dz
                    dm = jnp.minimum(dist_v[pl.ds(off, _L)], d)
                    dist_v[pl.ds(off, _L)] = dm
                    upd = dm > bval
                    return (jnp.where(upd, dm, bval),
                            jnp.where(upd, lane + off, bidx))

                bval, bidx = lax.fori_loop(
                    0, nchunks, chunk,
                    (jnp.full((_L,), -1.0, jnp.float32),
                     jnp.zeros((_L,), jnp.int32)), unroll=8)
                gmax = jnp.max(bval)
                nxt = jnp.min(jnp.where(bval == gmax, bidx,
                                        jnp.int32(n_points)))
                put(i, nxt)
                return nxt

            lax.fori_loop(1, n_samples, step, jnp.int32(0))
            pltpu.sync_copy(cx_v, out_h.at[3 * b + 0])
            pltpu.sync_copy(cy_v, out_h.at[3 * b + 1])
            pltpu.sync_copy(cz_v, out_h.at[3 * b + 2])
            if with_aux:
                pltpu.sync_copy(ao_v, aux_out_h.at[b])

    return functools.partial(
        pl.kernel, mesh=mesh, out_type=tuple(out_type) if with_aux
        else out_type[0], scratch_types=scratch,
        compiler_params=pltpu.CompilerParams(needs_layout_passes=False))(body)


def _fps_sc(xyz, n_samples, aux=None):
    n_points = xyz.shape[1]
    planes = jnp.transpose(xyz, (0, 2, 1)).reshape(3 * _B, n_points)
    if aux is None:
        out = _fps_sc_build(n_points, n_samples, False)(planes)
        outs = (out,)
    else:
        outs = _fps_sc_build(n_points, n_samples, True)(planes, aux)
    samp = outs[0].reshape(_B, 3, n_samples).transpose(0, 2, 1)
    if aux is None:
        return samp
    return samp, outs[1]


def _skel_compute(pts_ref, pT_ref, c_ref, d2_ref, nbr_ref, *, k, n):
    px = pT_ref[0, 0:1, :]
    py = pT_ref[0, 1:2, :]
    pz = pT_ref[0, 2:3, :]
    cen = c_ref[0]
    cx = cen[:, 0:1]
    cy = cen[:, 1:2]
    cz = cen[:, 2:3]
    dx = cx - px
    dy = cy - py
    dz = cz - pz
    d2 = dx * dx + dy * dy
    d2 = d2 + dz * dz
    mb = d2.shape[0]
    d2_ref[...] = d2
    iota = lax.broadcasted_iota(jnp.int32, (mb, n), 1)
    zero = jnp.float32(0.0)

    def select_pass(j, nsum):
        d2c = d2_ref[...]
        mval = jnp.min(d2c, axis=1, keepdims=True)
        idx = jnp.min(jnp.where(d2c == mval, iota, jnp.int32(n)),
                      axis=1, keepdims=True)
        oh = iota == idx
        d2_ref[...] = jnp.where(oh, jnp.float32(jnp.inf), d2c)
        nx = jnp.sum(jnp.where(oh, px, zero), axis=1, keepdims=True)
        ny = jnp.sum(jnp.where(oh, py, zero), axis=1, keepdims=True)
        nz = jnp.sum(jnp.where(oh, pz, zero), axis=1, keepdims=True)
        nbr = jnp.concatenate([nx, ny, nz], axis=1)
        nbr_ref[j] = nbr
        return nsum + nbr

    nsum = lax.fori_loop(0, k, select_pass, jnp.zeros((mb, 3), jnp.float32))
    newc = nsum / jnp.float32(k)

    def rad_pass(j, rsum):
        e = nbr_ref[j] - newc
        dd = e[:, 0:1] * e[:, 0:1] + e[:, 1:2] * e[:, 1:2]
        dd = dd + e[:, 2:3] * e[:, 2:3]
        return rsum + jnp.sqrt(dd + 1e-12)

    rad = lax.fori_loop(0, k, rad_pass,
                        jnp.zeros((mb, 1), jnp.float32)) / jnp.float32(k)
    return newc, rad


def _skel_body(pts_ref, pT_ref, c_ref, oc_ref, or_ref, d2_ref, nbr_ref,
               *, k, n):
    newc, rad = _skel_compute(pts_ref, pT_ref, c_ref, d2_ref, nbr_ref,
                              k=k, n=n)
    oc_ref[0] = newc
    or_ref[0] = rad


def _skel_tc(points, points_tp, centers, k, mb):
    b, n, _ = points.shape
    m = centers.shape[1]
    grid = (b, m // mb)
    return pl.pallas_call(
        functools.partial(_skel_body, k=k, n=n),
        grid=grid,
        in_specs=[
            pl.BlockSpec((1, n, 3), lambda i, j: (i, 0, 0)),
            pl.BlockSpec((1, 8, n), lambda i, j: (i, 0, 0)),
            pl.BlockSpec((1, mb, 3), lambda i, j: (i, j, 0)),
        ],
        out_specs=[
            pl.BlockSpec((1, mb, 3), lambda i, j: (i, j, 0)),
            pl.BlockSpec((1, mb, 1), lambda i, j: (i, j, 0)),
        ],
        out_shape=[
            jax.ShapeDtypeStruct((b, m, 3), jnp.float32),
            jax.ShapeDtypeStruct((b, m, 1), jnp.float32),
        ],
        scratch_shapes=[
            pltpu.VMEM((mb, n), jnp.float32),
            pltpu.VMEM((k, mb, 3), jnp.float32),
        ],
    )(points, points_tp, centers)


def _skel_final_body(pts_ref, pT_ref, c_ref, rg_ref, u_ref,
                     oc_ref, or_ref, os_ref, d2_ref, nbr_ref, *, k, n):
    newc, rad = _skel_compute(pts_ref, pT_ref, c_ref, d2_ref, nbr_ref,
                              k=k, n=n)
    oc_ref[0] = newc
    radf = (rg_ref[0] + rad) * jnp.float32(_SCALE)
    or_ref[0] = radf
    cexp = jnp.concatenate([newc] * _SP, axis=1)
    uexp = u_ref[0:1, :]
    os_ref[0] = cexp + radf * uexp


def _fibonacci_sphere(npts):
    i = jnp.arange(npts, dtype=jnp.float32)
    golden = jnp.pi * (3.0 - jnp.sqrt(5.0))
    y = 1.0 - 2.0 * (i + 0.5) / npts
    r = jnp.sqrt(jnp.clip(1.0 - y * y, 0.0, 1.0))
    theta = golden * i
    return jnp.stack([r * jnp.cos(theta), y, r * jnp.sin(theta)], axis=-1)


def _skel_final_tc(points, points_tp, centers, rgath, k, mb):
    b, n, _ = points.shape
    m = centers.shape[1]
    grid = (b, m // mb)
    uflat = jnp.tile(_fibonacci_sphere(_SP).reshape(1, 3 * _SP), (8, 1))
    return pl.pallas_call(
        functools.partial(_skel_final_body, k=k, n=n),
        grid=grid,
        in_specs=[
            pl.BlockSpec((1, n, 3), lambda i, j: (i, 0, 0)),
            pl.BlockSpec((1, 8, n), lambda i, j: (i, 0, 0)),
            pl.BlockSpec((1, mb, 3), lambda i, j: (i, j, 0)),
            pl.BlockSpec((1, mb, 1), lambda i, j: (i, j, 0)),
            pl.BlockSpec((8, 3 * _SP), lambda i, j: (0, 0)),
        ],
        out_specs=[
            pl.BlockSpec((1, mb, 3), lambda i, j: (i, j, 0)),
            pl.BlockSpec((1, mb, 1), lambda i, j: (i, j, 0)),
            pl.BlockSpec((1, mb, 3 * _SP), lambda i, j: (i, j, 0)),
        ],
        out_shape=[
            jax.ShapeDtypeStruct((b, m, 3), jnp.float32),
            jax.ShapeDtypeStruct((b, m, 1), jnp.float32),
            jax.ShapeDtypeStruct((b, m, 3 * _SP), jnp.float32),
        ],
        scratch_shapes=[
            pltpu.VMEM((mb, n), jnp.float32),
            pltpu.VMEM((k, mb, 3), jnp.float32),
        ],
    )(points, points_tp, centers, rgath, uflat)


def _pad_tp(points):
    tp = jnp.transpose(points, (0, 2, 1))
    return jnp.pad(tp, ((0, 0), (0, 5), (0, 0)))


def kernel(xyz):
    c1 = _fps_sc(xyz, _SKP * 4)
    xyz_tp = _pad_tp(xyz)
    nc1, _ = _skel_tc(xyz, xyz_tp, c1, _K1, 256)
    xyz2, rad1 = _skel_tc(xyz, xyz_tp, nc1, _K1, 256)
    c2, rg = _fps_sc(xyz2, _SKP, aux=rad1.reshape(_B, -1))
    xyz2_tp = _pad_tp(xyz2)
    nc2, _ = _skel_tc(xyz2, xyz2_tp, c2, _K2, 256)
    centers, radius, sflat = _skel_final_tc(
        xyz2, xyz2_tp, nc2, rg.reshape(_B, _SKP, 1), _K2, 256)
    sphere = sflat.reshape(_B, _SKP, _SP, 3)
    return centers, sphere, radius
